# baseline (speedup 1.0000x reference)
import functools

import jax
import jax.numpy as jnp
from jax import lax
from jax.experimental import pallas as pl
from jax.experimental.pallas import tpu as pltpu
from jax.experimental.pallas import tpu_sc as plsc

E = 8
T = 4096
D_IN = 2048
D_OUT = 2048

BM = 256
M_PAD = 5888
N_TILES_M = M_PAD // BM

_info = plsc.get_sparse_core_info()
NC, NS, L = _info.num_cores, _info.num_subcores, _info.num_lanes
NW = NC * NS
TB = T // NW
CH = TB // L
D2 = D_IN // 2

_mesh = lambda: plsc.VectorSubcoreMesh(core_axis_name="c", subcore_axis_name="s")


def _wid():
    return lax.axis_index("s") * NC + lax.axis_index("c")


def _lane_iota():
    return lax.iota(jnp.int32, L)


def _bcast_last(v):
    last = jnp.full((L,), L - 1, jnp.int32)
    return v.at[last].get(mode="promise_in_bounds")


def _popcount(m):
    return _bcast_last(plsc.cumsum(jnp.where(m, 1, 0)))


@functools.partial(
    pl.kernel,
    mesh=_mesh(),
    out_type=jax.ShapeDtypeStruct((NW, L), jnp.int32),
    scratch_types=[
        pltpu.VMEM((CH, L), jnp.int32),
        pltpu.VMEM((L,), jnp.int32),
    ],
    compiler_params=pltpu.CompilerParams(needs_layout_passes=False),
)
def _sc_count(ids_hbm, grid_hbm, ids_v, cnt_v):
    wid = _wid()
    lanes = _lane_iota()
    pltpu.sync_copy(ids_hbm.at[pl.ds(wid * CH, CH)], ids_v)
    cnt = jnp.zeros((L,), jnp.int32)
    for c in range(CH):
        ids = ids_v[c]
        for e in range(E):
            pc = _popcount(ids == e)
            cnt = cnt + jnp.where(lanes == e, pc, 0)
    cnt_v[...] = cnt
    pltpu.sync_copy(cnt_v, grid_hbm.at[wid])


@functools.partial(
    pl.kernel,
    mesh=_mesh(),
    out_type=[
        jax.ShapeDtypeStruct((M_PAD, D_IN), jnp.float32),
        jax.ShapeDtypeStruct((T // L, L), jnp.int32),
    ],
    scratch_types=[
        pltpu.VMEM((CH, L), jnp.int32),
        pltpu.VMEM((CH, L), jnp.int32),
        pltpu.VMEM((NW, L), jnp.int32),
        pltpu.VMEM((L, D_IN), jnp.float32),
        pltpu.VMEM((L, D_IN), jnp.float32),
        pltpu.SemaphoreType.DMA,
        pltpu.SemaphoreType.DMA,
        pltpu.SemaphoreType.DMA,
        pltpu.SemaphoreType.DMA,
    ],
    compiler_params=pltpu.CompilerParams(needs_layout_passes=False),
)
def _sc_dispatch(ids_hbm, grid_hbm, tokens_hbm, xs_hbm, dest_hbm,
                 ids_v, dest_v, grid_v, rows0_v, rows1_v,
                 sl0, sl1, ss0, ss1):
    wid = _wid()
    lanes = _lane_iota()
    pltpu.sync_copy(ids_hbm.at[pl.ds(wid * CH, CH)], ids_v)
    pltpu.sync_copy(grid_hbm, grid_v)

    rows = (rows0_v, rows1_v)
    sl = (sl0, sl1)
    ss = (ss0, ss1)

    def tok_slice(c):
        return tokens_hbm.at[pl.ds(wid * TB + c * L, L)]

    ld = [None] * CH
    ld[0] = pltpu.async_copy(tok_slice(0), rows[0], sl[0])
    ld[1] = pltpu.async_copy(tok_slice(1), rows[1], sl[1])

    widv = jnp.zeros((L,), jnp.int32) + wid
    total = jnp.zeros((L,), jnp.int32)
    before = jnp.zeros((L,), jnp.int32)
    for w2 in range(NW):
        row = grid_v[w2]
        total = total + row
        before = before + jnp.where(jnp.full((L,), w2, jnp.int32) < widv, row, 0)

    aligned = jnp.bitwise_and(total + (BM - 1), -BM)
    off = plsc.cumsum(aligned) - aligned
    base = off + before

    carry = jnp.zeros((L,), jnp.int32)
    for c in range(CH):
        ids = ids_v[c]
        base_tok = (base + carry).at[ids].get(mode="promise_in_bounds")
        prior = jnp.zeros((L,), jnp.int32)
        for e in range(E):
            m = ids == e
            cs = plsc.cumsum(jnp.where(m, 1, 0))
            prior = jnp.where(m, cs - 1, prior)
            pc = _bcast_last(cs)
            carry = carry + jnp.where(lanes == e, pc, 0)
        dest_v[c] = base_tok + prior

    pltpu.sync_copy(dest_v, dest_hbm.at[pl.ds(wid * CH, CH)])

    for c in range(CH):
        b = c & 1
        ld[c].wait()
        sc = pltpu.async_copy(rows[b], xs_hbm.at[dest_v.at[c]], ss[b])
        sc.wait()
        if c + 2 < CH:
            ld[c + 2] = pltpu.async_copy(tok_slice(c + 2), rows[b], sl[b])


@functools.partial(
    pl.kernel,
    mesh=_mesh(),
    out_type=jax.ShapeDtypeStruct((T, D_OUT), jnp.float32),
    scratch_types=[
        pltpu.VMEM((CH, L), jnp.int32),
        pltpu.VMEM((L, D_OUT), jnp.float32),
        pltpu.VMEM((L, D_OUT), jnp.float32),
        pltpu.SemaphoreType.DMA,
        pltpu.SemaphoreType.DMA,
        pltpu.SemaphoreType.DMA,
        pltpu.SemaphoreType.DMA,
    ],
    compiler_params=pltpu.CompilerParams(needs_layout_passes=False),
)
def _sc_combine(y_hbm, dest_hbm, out_hbm, dest_v, rows0_v, rows1_v,
                sg0, sg1, st0, st1):
    wid = _wid()
    pltpu.sync_copy(dest_hbm.at[pl.ds(wid * CH, CH)], dest_v)
    rows = (rows0_v, rows1_v)
    sg = (sg0, sg1)
    st = (st0, st1)

    def out_slice(c):
        return out_hbm.at[pl.ds(wid * TB + c * L, L)]

    gt = [None] * CH
    gt[0] = pltpu.async_copy(y_hbm.at[dest_v.at[0]], rows[0], sg[0])
    gt[1] = pltpu.async_copy(y_hbm.at[dest_v.at[1]], rows[1], sg[1])
    for c in range(CH):
        b = c & 1
        gt[c].wait()
        so = pltpu.async_copy(rows[b], out_slice(c), st[b])
        so.wait()
        if c + 2 < CH:
            gt[c + 2] = pltpu.async_copy(y_hbm.at[dest_v.at[c + 2]], rows[b], sg[b])


BN = D_OUT


def _gmm_body(te_ref, x_ref, w_ref, o_ref):
    o_ref[...] = lax.dot_general(
        x_ref[...].astype(jnp.bfloat16), w_ref[0],
        (((1,), (0,)), ((), ())),
        preferred_element_type=jnp.float32)


def _gmm(tile_expert, xs, weight):
    grid_spec = pltpu.PrefetchScalarGridSpec(
        num_scalar_prefetch=1,
        grid=(D_OUT // BN, N_TILES_M),
        in_specs=[
            pl.BlockSpec((BM, D_IN), lambda j, i, te: (i, 0)),
            pl.BlockSpec((1, D_IN, BN), lambda j, i, te: (te[i], 0, j)),
        ],
        out_specs=pl.BlockSpec((BM, BN), lambda j, i, te: (i, j)),
    )
    return pl.pallas_call(
        _gmm_body,
        grid_spec=grid_spec,
        out_shape=jax.ShapeDtypeStruct((M_PAD, D_OUT), jnp.float32),
    )(tile_expert, xs, weight)


def kernel(tokens, exp_ids, weight):
    w_bf = weight.astype(jnp.bfloat16)
    ids2d = exp_ids.reshape(T // L, L)
    counts_grid = _sc_count(ids2d)
    xs, dest2d = _sc_dispatch(ids2d, counts_grid, tokens)

    total = jnp.sum(counts_grid, axis=0)[:E]
    tiles_per_e = (total + (BM - 1)) // BM
    tile_expert = jnp.repeat(
        jnp.arange(E, dtype=jnp.int32), tiles_per_e,
        total_repeat_length=N_TILES_M)

    y = _gmm(tile_expert, xs, w_bf)
    return _sc_combine(y, dest2d)


# speedup vs baseline: 1.1768x; 1.1768x over previous
import functools

import jax
import jax.numpy as jnp
from jax import lax
from jax.experimental import pallas as pl
from jax.experimental.pallas import tpu as pltpu
from jax.experimental.pallas import tpu_sc as plsc

E = 8
T = 4096
D_IN = 2048
D_OUT = 2048

BM = 256
M_PAD = 5888
N_TILES_M = M_PAD // BM

_info = plsc.get_sparse_core_info()
NC, NS, L = _info.num_cores, _info.num_subcores, _info.num_lanes
NW = NC * NS
TB = T // NW
CH = TB // L
D2 = D_IN // 2

_mesh = lambda: plsc.VectorSubcoreMesh(core_axis_name="c", subcore_axis_name="s")


def _wid():
    return lax.axis_index("s") * NC + lax.axis_index("c")


def _lane_iota():
    return lax.iota(jnp.int32, L)


def _bcast_last(v):
    last = jnp.full((L,), L - 1, jnp.int32)
    return v.at[last].get(mode="promise_in_bounds")


def _popcount(m):
    return _bcast_last(plsc.cumsum(jnp.where(m, 1, 0)))


@functools.partial(
    pl.kernel,
    mesh=_mesh(),
    out_type=jax.ShapeDtypeStruct((NW, L), jnp.int32),
    scratch_types=[
        pltpu.VMEM((CH, L), jnp.int32),
        pltpu.VMEM((L,), jnp.int32),
    ],
    compiler_params=pltpu.CompilerParams(needs_layout_passes=False),
)
def _sc_count(ids_hbm, grid_hbm, ids_v, cnt_v):
    wid = _wid()
    lanes = _lane_iota()
    pltpu.sync_copy(ids_hbm.at[pl.ds(wid * CH, CH)], ids_v)
    cnt = jnp.zeros((L,), jnp.int32)
    for c in range(CH):
        ids = ids_v[c]
        for e in range(E):
            pc = _popcount(ids == e)
            cnt = cnt + jnp.where(lanes == e, pc, 0)
    cnt_v[...] = cnt
    pltpu.sync_copy(cnt_v, grid_hbm.at[wid])


@functools.partial(
    pl.kernel,
    mesh=_mesh(),
    out_type=[
        jax.ShapeDtypeStruct((M_PAD, D_IN), jnp.float32),
        jax.ShapeDtypeStruct((T // L, L), jnp.int32),
    ],
    scratch_types=[
        pltpu.VMEM((CH, L), jnp.int32),
        pltpu.VMEM((CH, L), jnp.int32),
        pltpu.VMEM((NW, L), jnp.int32),
        pltpu.VMEM((L, D_IN), jnp.float32),
        pltpu.VMEM((L, D_IN), jnp.float32),
        pltpu.SemaphoreType.DMA,
        pltpu.SemaphoreType.DMA,
        pltpu.SemaphoreType.DMA,
        pltpu.SemaphoreType.DMA,
    ],
    compiler_params=pltpu.CompilerParams(needs_layout_passes=False),
)
def _sc_dispatch(ids_hbm, grid_hbm, tokens_hbm, xs_hbm, dest_hbm,
                 ids_v, dest_v, grid_v, rows0_v, rows1_v,
                 sl0, sl1, ss0, ss1):
    wid = _wid()
    lanes = _lane_iota()
    pltpu.sync_copy(ids_hbm.at[pl.ds(wid * CH, CH)], ids_v)
    pltpu.sync_copy(grid_hbm, grid_v)

    rows = (rows0_v, rows1_v)
    sl = (sl0, sl1)
    ss = (ss0, ss1)

    def tok_slice(c):
        return tokens_hbm.at[pl.ds(wid * TB + c * L, L)]

    ld = [None] * CH
    ld[0] = pltpu.async_copy(tok_slice(0), rows[0], sl[0])
    ld[1] = pltpu.async_copy(tok_slice(1), rows[1], sl[1])

    widv = jnp.zeros((L,), jnp.int32) + wid
    total = jnp.zeros((L,), jnp.int32)
    before = jnp.zeros((L,), jnp.int32)
    for w2 in range(NW):
        row = grid_v[w2]
        total = total + row
        before = before + jnp.where(jnp.full((L,), w2, jnp.int32) < widv, row, 0)

    aligned = jnp.bitwise_and(total + (BM - 1), -BM)
    off = plsc.cumsum(aligned) - aligned
    base = off + before

    carry = jnp.zeros((L,), jnp.int32)
    for c in range(CH):
        ids = ids_v[c]
        base_tok = (base + carry).at[ids].get(mode="promise_in_bounds")
        prior = jnp.zeros((L,), jnp.int32)
        for e in range(E):
            m = ids == e
            cs = plsc.cumsum(jnp.where(m, 1, 0))
            prior = jnp.where(m, cs - 1, prior)
            pc = _bcast_last(cs)
            carry = carry + jnp.where(lanes == e, pc, 0)
        dest_v[c] = base_tok + prior

    pltpu.sync_copy(dest_v, dest_hbm.at[pl.ds(wid * CH, CH)])

    for c in range(CH):
        b = c & 1
        ld[c].wait()
        sc = pltpu.async_copy(rows[b], xs_hbm.at[dest_v.at[c]], ss[b])
        sc.wait()
        if c + 2 < CH:
            ld[c + 2] = pltpu.async_copy(tok_slice(c + 2), rows[b], sl[b])


@functools.partial(
    pl.kernel,
    mesh=_mesh(),
    out_type=jax.ShapeDtypeStruct((T, D_OUT), jnp.float32),
    scratch_types=[
        pltpu.VMEM((CH, L), jnp.int32),
        pltpu.VMEM((L, D_OUT), jnp.float32),
        pltpu.VMEM((L, D_OUT), jnp.float32),
        pltpu.SemaphoreType.DMA,
        pltpu.SemaphoreType.DMA,
        pltpu.SemaphoreType.DMA,
        pltpu.SemaphoreType.DMA,
    ],
    compiler_params=pltpu.CompilerParams(needs_layout_passes=False),
)
def _sc_combine(y_hbm, dest_hbm, out_hbm, dest_v, rows0_v, rows1_v,
                sg0, sg1, st0, st1):
    wid = _wid()
    pltpu.sync_copy(dest_hbm.at[pl.ds(wid * CH, CH)], dest_v)
    rows = (rows0_v, rows1_v)
    sg = (sg0, sg1)
    st = (st0, st1)

    def out_slice(c):
        return out_hbm.at[pl.ds(wid * TB + c * L, L)]

    gt = [None] * CH
    gt[0] = pltpu.async_copy(y_hbm.at[dest_v.at[0]], rows[0], sg[0])
    gt[1] = pltpu.async_copy(y_hbm.at[dest_v.at[1]], rows[1], sg[1])
    for c in range(CH):
        b = c & 1
        gt[c].wait()
        so = pltpu.async_copy(rows[b], out_slice(c), st[b])
        so.wait()
        if c + 2 < CH:
            gt[c + 2] = pltpu.async_copy(y_hbm.at[dest_v.at[c + 2]], rows[b], sg[b])


BN = D_OUT


def _gmm_body(te_ref, x_ref, w_ref, o_ref, wbf_ref):
    i = pl.program_id(1)

    @pl.when((i == 0) | (te_ref[i] != te_ref[jnp.maximum(i - 1, 0)]))
    def _():
        wbf_ref[...] = w_ref[0].astype(jnp.bfloat16)

    o_ref[...] = lax.dot_general(
        x_ref[...].astype(jnp.bfloat16), wbf_ref[...],
        (((1,), (0,)), ((), ())),
        preferred_element_type=jnp.float32)


def _gmm(tile_expert, xs, weight):
    grid_spec = pltpu.PrefetchScalarGridSpec(
        num_scalar_prefetch=1,
        grid=(D_OUT // BN, N_TILES_M),
        in_specs=[
            pl.BlockSpec((BM, D_IN), lambda j, i, te: (i, 0)),
            pl.BlockSpec((1, D_IN, BN), lambda j, i, te: (te[i], 0, j)),
        ],
        out_specs=pl.BlockSpec((BM, BN), lambda j, i, te: (i, j)),
        scratch_shapes=[pltpu.VMEM((D_IN, BN), jnp.bfloat16)],
    )
    return pl.pallas_call(
        _gmm_body,
        grid_spec=grid_spec,
        out_shape=jax.ShapeDtypeStruct((M_PAD, D_OUT), jnp.float32),
    )(tile_expert, xs, weight)


def kernel(tokens, exp_ids, weight):
    ids2d = exp_ids.reshape(T // L, L)
    counts_grid = _sc_count(ids2d)
    xs, dest2d = _sc_dispatch(ids2d, counts_grid, tokens)

    total = jnp.sum(counts_grid, axis=0)[:E]
    tiles_per_e = (total + (BM - 1)) // BM
    tile_expert = jnp.repeat(
        jnp.arange(E, dtype=jnp.int32), tiles_per_e,
        total_repeat_length=N_TILES_M)

    y = _gmm(tile_expert, xs, weight)
    return _sc_combine(y, dest2d)
